# baseline (speedup 1.0000x reference)
import jax
import jax.numpy as jnp
from jax.experimental import pallas as pl

N_REGIONS = 6
BOUND = 2.5


def _pwlu_tc_kernel(x_ref, a_ref, b_ref, out_ref):
    x = x_ref[...]
    s = x * (0.5 * N_REGIONS / BOUND) + (0.5 * N_REGIONS)
    a = jnp.full_like(s, a_ref[0, 0, 0])
    b = jnp.full_like(s, b_ref[0, 0, 0])
    for j in range(1, N_REGIONS):
        m = s >= float(j)
        a = jnp.where(m, a_ref[0, 0, j], a)
        b = jnp.where(m, b_ref[0, 0, j], b)
    out_ref[...] = a + b * s


def kernel(x, points):
    B, C, H, W = x.shape

    slopes = points[:, 1:] - points[:, :-1]
    intercepts = points[:, :-1] - slopes * jnp.arange(
        N_REGIONS, dtype=points.dtype
    )[None, :]
    a_t = intercepts.reshape(C, 1, N_REGIONS)
    b_t = slopes.reshape(C, 1, N_REGIONS)

    HBLK = 56
    grid = (C, H // HBLK)
    out = pl.pallas_call(
        _pwlu_tc_kernel,
        grid=grid,
        in_specs=[
            pl.BlockSpec((B, 1, HBLK, W), lambda c, h: (0, c, h, 0)),
            pl.BlockSpec((1, 1, N_REGIONS), lambda c, h: (c, 0, 0)),
            pl.BlockSpec((1, 1, N_REGIONS), lambda c, h: (c, 0, 0)),
        ],
        out_specs=pl.BlockSpec((B, 1, HBLK, W), lambda c, h: (0, c, h, 0)),
        out_shape=jax.ShapeDtypeStruct((B, C, H, W), x.dtype),
    )(x, a_t, b_t)
    return out
